# Initial kernel scaffold; baseline (speedup 1.0000x reference)
import functools

import jax
import jax.numpy as jnp
from jax import lax
from jax.experimental import pallas as pl
from jax.experimental.pallas import tpu as pltpu
from jax.experimental.pallas import tpu_sc as plsc

NC = 2
NS = 16
LANES = 16

D = 128


def _mm_body(x_ref, w_ref, b_ref, o_ref):
    o_ref[...] = (
        jnp.dot(x_ref[...], w_ref[...], preferred_element_type=jnp.float32)
        + b_ref[...]
    )


def _matmul(x, w, b, bm):
    m, k = x.shape
    f = w.shape[1]
    return pl.pallas_call(
        _mm_body,
        grid=(m // bm,),
        in_specs=[
            pl.BlockSpec((bm, k), lambda i: (i, 0)),
            pl.BlockSpec((k, f), lambda i: (0, 0)),
            pl.BlockSpec((1, f), lambda i: (0, 0)),
        ],
        out_specs=pl.BlockSpec((bm, f), lambda i: (i, 0)),
        out_shape=jax.ShapeDtypeStruct((m, f), jnp.float32),
    )(x, w, b.reshape(1, f))


def _mm2_body(p_ref, w_ref, b_ref, o_ref):
    h = jax.nn.relu(p_ref[0] + p_ref[1])
    o_ref[...] = (
        jnp.dot(h, w_ref[...], preferred_element_type=jnp.float32) + b_ref[...]
    )


def _relu_partials_matmul(p, w, b, bm):
    _, m, k = p.shape
    f = w.shape[1]
    return pl.pallas_call(
        _mm2_body,
        grid=(m // bm,),
        in_specs=[
            pl.BlockSpec((2, bm, k), lambda i: (0, i, 0)),
            pl.BlockSpec((k, f), lambda i: (0, 0)),
            pl.BlockSpec((1, f), lambda i: (0, 0)),
        ],
        out_specs=pl.BlockSpec((bm, f), lambda i: (i, 0)),
        out_shape=jax.ShapeDtypeStruct((m, f), jnp.float32),
    )(p, w, b.reshape(1, f))


def _gph_body(p_ref, fw_ref, o_ref):
    o_ref[...] = (p_ref[0] + p_ref[1]) * fw_ref[0]


def _scaled_partials(p, fw, bm):
    _, m, k = p.shape
    return pl.pallas_call(
        _gph_body,
        grid=(m // bm,),
        in_specs=[
            pl.BlockSpec((2, bm, k), lambda i: (0, i, 0)),
            pl.BlockSpec(memory_space=pltpu.SMEM),
        ],
        out_specs=pl.BlockSpec((bm, k), lambda i: (i, 0)),
        out_shape=jax.ShapeDtypeStruct((m, k), jnp.float32),
    )(p, fw)


def _proj_body(img_ref, rel_ref, att_ref, name_ref, char_ref,
               iw_ref, ib_ref, rw_ref, rb_ref, aw_ref, ab_ref,
               nw_ref, nb_ref, cw_ref, cb_ref, fw_ref, o_ref):
    def mm(x_ref, w_ref, b_ref, s):
        return (
            jnp.dot(x_ref[...], w_ref[...], preferred_element_type=jnp.float32)
            + b_ref[...]
        ) * s

    o_ref[:, 0, :] = mm(rel_ref, rw_ref, rb_ref, fw_ref[1])
    o_ref[:, 1, :] = mm(att_ref, aw_ref, ab_ref, fw_ref[2])
    o_ref[:, 2, :] = mm(img_ref, iw_ref, ib_ref, fw_ref[3])
    o_ref[:, 3, :] = mm(name_ref, nw_ref, nb_ref, fw_ref[4])
    o_ref[:, 4, :] = mm(char_ref, cw_ref, cb_ref, fw_ref[5])


def _projections(img, rel, att, name, char, iw, ib, rw, rb, aw, ab,
                 nw, nb, cw, cb, fw, bm):
    m = img.shape[0]
    f = iw.shape[1]

    def row_spec(x):
        k = x.shape[1]
        return pl.BlockSpec((bm, k), lambda i: (i, 0))

    def w_spec(w):
        k = w.shape[0]
        return pl.BlockSpec((k, f), lambda i: (0, 0))

    b_spec = pl.BlockSpec((1, f), lambda i: (0, 0))
    return pl.pallas_call(
        _proj_body,
        grid=(m // bm,),
        in_specs=[
            row_spec(img), row_spec(rel), row_spec(att), row_spec(name),
            row_spec(char),
            w_spec(iw), b_spec, w_spec(rw), b_spec, w_spec(aw), b_spec,
            w_spec(nw), b_spec, w_spec(cw), b_spec,
            pl.BlockSpec(memory_space=pltpu.SMEM),
        ],
        out_specs=pl.BlockSpec((bm, 5, f), lambda i: (i, 0, 0)),
        out_shape=jax.ShapeDtypeStruct((m, 5, f), jnp.float32),
    )(img, rel, att, name, char,
      iw, ib.reshape(1, f), rw, rb.reshape(1, f), aw, ab.reshape(1, f),
      nw, nb.reshape(1, f), cw, cb.reshape(1, f), fw)


def _spmm_sc(src, dst, table, chunk=80):
    e = src.shape[0]
    n, d = table.shape
    epw = e // (NC * NS)
    n_chunks = epw // chunk
    assert epw % chunk == 0 and chunk % 8 == 0
    rows_pt = n // NS
    zr = 125
    assert rows_pt % zr == 0

    mesh = plsc.VectorSubcoreMesh(
        core_axis_name="c", subcore_axis_name="s",
        num_cores=NC, num_subcores=NS)

    @functools.partial(
        pl.kernel,
        mesh=mesh,
        out_type=jax.ShapeDtypeStruct((NC, n, d), jnp.float32),
        scratch_types=[
            pltpu.VMEM((chunk,), jnp.int32),
            pltpu.VMEM((chunk,), jnp.int32),
            pltpu.VMEM((chunk, d), jnp.float32),
            pltpu.VMEM((zr, d), jnp.float32),
            pltpu.VMEM_SHARED((n, d), jnp.float32),
            pltpu.SemaphoreType.DMA,
        ],
    )
    def k(src_hbm, dst_hbm, table_hbm, out_hbm, sidx, didx, rows, zbuf,
          accum, sem):
        c = lax.axis_index("c")
        s = lax.axis_index("s")

        def zfill(i, _):
            zbuf[i // (d // LANES),
                 pl.ds((i % (d // LANES)) * LANES, LANES)] = (
                jnp.zeros((LANES,), jnp.float32))
            return 0
        lax.fori_loop(0, zr * (d // LANES), zfill, 0)
        row0 = s * rows_pt
        for r in range(0, rows_pt, zr):
            pltpu.sync_copy(zbuf, accum.at[pl.ds(row0 + r, zr)])
        plsc.subcore_barrier()

        base0 = (c * NS + s) * epw
        def body(j, _):
            b = base0 + j * chunk
            pltpu.sync_copy(src_hbm.at[pl.ds(b, chunk)], sidx)
            pltpu.sync_copy(dst_hbm.at[pl.ds(b, chunk)], didx)
            pltpu.async_copy(table_hbm.at[sidx], rows, sem).wait()
            pltpu.sync_copy(rows, accum.at[didx], add=True)
            return 0
        lax.fori_loop(0, n_chunks, body, 0)
        plsc.subcore_barrier()

        pltpu.sync_copy(accum.at[pl.ds(row0, rows_pt)],
                        out_hbm.at[c, pl.ds(row0, rows_pt)])

    return k(src, dst, table)


def kernel(input_idx, edge_index, img_features, rel_features, att_features,
           name_features, char_features, entity_emb, gc1_w, gc1_b, gc2_w,
           gc2_b, img_w, img_b, rel_w, rel_b, att_w, att_b, name_w, name_b,
           char_w, char_b, fusion_weight):
    n = entity_emb.shape[0]
    x = jnp.take(entity_emb, input_idx, axis=0)
    src = edge_index[0]
    dst = edge_index[1]

    z1 = _matmul(x, gc1_w, gc1_b, bm=1000)
    p1 = _spmm_sc(src, dst, z1)
    z2 = _relu_partials_matmul(p1, gc2_w, gc2_b, bm=1000)
    p2 = _spmm_sc(src, dst, z2)
    gph = _scaled_partials(p2, fusion_weight, bm=1000)

    proj = _projections(
        img_features, rel_features, att_features, name_features,
        char_features, img_w, img_b, rel_w, rel_b, att_w, att_b,
        name_w, name_b, char_w, char_b, fusion_weight, bm=500)

    return jnp.concatenate([gph, proj.reshape(n, -1)], axis=-1)


# speedup vs baseline: 4.2498x; 4.2498x over previous
import functools

import jax
import jax.numpy as jnp
from jax import lax
from jax.experimental import pallas as pl
from jax.experimental.pallas import tpu as pltpu
from jax.experimental.pallas import tpu_sc as plsc

NC = 2
NS = 16
LANES = 16

D = 128


def _mm_body(x_ref, w_ref, b_ref, o_ref):
    o_ref[...] = (
        jnp.dot(x_ref[...], w_ref[...], preferred_element_type=jnp.float32)
        + b_ref[...]
    )


def _matmul(x, w, b, bm):
    m, k = x.shape
    f = w.shape[1]
    return pl.pallas_call(
        _mm_body,
        grid=(m // bm,),
        in_specs=[
            pl.BlockSpec((bm, k), lambda i: (i, 0)),
            pl.BlockSpec((k, f), lambda i: (0, 0)),
            pl.BlockSpec((1, f), lambda i: (0, 0)),
        ],
        out_specs=pl.BlockSpec((bm, f), lambda i: (i, 0)),
        out_shape=jax.ShapeDtypeStruct((m, f), jnp.float32),
    )(x, w, b.reshape(1, f))


def _mm2_body(p_ref, w_ref, b_ref, o_ref):
    h = jax.nn.relu(p_ref[0] + p_ref[1])
    o_ref[...] = (
        jnp.dot(h, w_ref[...], preferred_element_type=jnp.float32) + b_ref[...]
    )


def _relu_partials_matmul(p, w, b, bm):
    _, m, k = p.shape
    f = w.shape[1]
    return pl.pallas_call(
        _mm2_body,
        grid=(m // bm,),
        in_specs=[
            pl.BlockSpec((2, bm, k), lambda i: (0, i, 0)),
            pl.BlockSpec((k, f), lambda i: (0, 0)),
            pl.BlockSpec((1, f), lambda i: (0, 0)),
        ],
        out_specs=pl.BlockSpec((bm, f), lambda i: (i, 0)),
        out_shape=jax.ShapeDtypeStruct((m, f), jnp.float32),
    )(p, w, b.reshape(1, f))


def _gph_body(p_ref, fw_ref, o_ref):
    o_ref[...] = (p_ref[0] + p_ref[1]) * fw_ref[0]


def _scaled_partials(p, fw, bm):
    _, m, k = p.shape
    return pl.pallas_call(
        _gph_body,
        grid=(m // bm,),
        in_specs=[
            pl.BlockSpec((2, bm, k), lambda i: (0, i, 0)),
            pl.BlockSpec(memory_space=pltpu.SMEM),
        ],
        out_specs=pl.BlockSpec((bm, k), lambda i: (i, 0)),
        out_shape=jax.ShapeDtypeStruct((m, k), jnp.float32),
    )(p, fw)


def _proj_body(img_ref, rel_ref, att_ref, name_ref, char_ref,
               iw_ref, ib_ref, rw_ref, rb_ref, aw_ref, ab_ref,
               nw_ref, nb_ref, cw_ref, cb_ref, fw_ref, o_ref):
    def mm(x_ref, w_ref, b_ref, s):
        return (
            jnp.dot(x_ref[...], w_ref[...], preferred_element_type=jnp.float32)
            + b_ref[...]
        ) * s

    o_ref[:, 0, :] = mm(rel_ref, rw_ref, rb_ref, fw_ref[1])
    o_ref[:, 1, :] = mm(att_ref, aw_ref, ab_ref, fw_ref[2])
    o_ref[:, 2, :] = mm(img_ref, iw_ref, ib_ref, fw_ref[3])
    o_ref[:, 3, :] = mm(name_ref, nw_ref, nb_ref, fw_ref[4])
    o_ref[:, 4, :] = mm(char_ref, cw_ref, cb_ref, fw_ref[5])


def _projections(img, rel, att, name, char, iw, ib, rw, rb, aw, ab,
                 nw, nb, cw, cb, fw, bm):
    m = img.shape[0]
    f = iw.shape[1]

    def row_spec(x):
        k = x.shape[1]
        return pl.BlockSpec((bm, k), lambda i: (i, 0))

    def w_spec(w):
        k = w.shape[0]
        return pl.BlockSpec((k, f), lambda i: (0, 0))

    b_spec = pl.BlockSpec((1, f), lambda i: (0, 0))
    return pl.pallas_call(
        _proj_body,
        grid=(m // bm,),
        in_specs=[
            row_spec(img), row_spec(rel), row_spec(att), row_spec(name),
            row_spec(char),
            w_spec(iw), b_spec, w_spec(rw), b_spec, w_spec(aw), b_spec,
            w_spec(nw), b_spec, w_spec(cw), b_spec,
            pl.BlockSpec(memory_space=pltpu.SMEM),
        ],
        out_specs=pl.BlockSpec((bm, 5, f), lambda i: (i, 0, 0)),
        out_shape=jax.ShapeDtypeStruct((m, 5, f), jnp.float32),
    )(img, rel, att, name, char,
      iw, ib.reshape(1, f), rw, rb.reshape(1, f), aw, ab.reshape(1, f),
      nw, nb.reshape(1, f), cw, cb.reshape(1, f), fw)


def _spmm_sc(src, dst, table, chunk=80):
    e = src.shape[0]
    n, d = table.shape
    epw = e // (NC * NS)
    n_chunks = epw // chunk
    assert epw % chunk == 0 and chunk % 8 == 0
    rows_pt = (n // NS) // 8 * 8
    tail = n - rows_pt * NS
    zr = rows_pt // 2
    assert rows_pt % 2 == 0 and tail % 8 == 0 and tail <= zr

    mesh = plsc.VectorSubcoreMesh(
        core_axis_name="c", subcore_axis_name="s",
        num_cores=NC, num_subcores=NS)

    @functools.partial(
        pl.kernel,
        mesh=mesh,
        out_type=jax.ShapeDtypeStruct((NC, n, d), jnp.float32),
        scratch_types=[
            pltpu.VMEM((chunk,), jnp.int32),
            pltpu.VMEM((chunk,), jnp.int32),
            pltpu.VMEM((chunk, d), jnp.float32),
            pltpu.VMEM((zr, d), jnp.float32),
            pltpu.VMEM_SHARED((n, d), jnp.float32),
            pltpu.SemaphoreType.DMA,
        ],
    )
    def k(src_hbm, dst_hbm, table_hbm, out_hbm, sidx, didx, rows, zbuf,
          accum, sem):
        c = lax.axis_index("c")
        s = lax.axis_index("s")

        def zfill(i, _):
            zbuf[i // (d // LANES),
                 pl.ds((i % (d // LANES)) * LANES, LANES)] = (
                jnp.zeros((LANES,), jnp.float32))
            return 0
        lax.fori_loop(0, zr * (d // LANES), zfill, 0)
        row0 = s * rows_pt
        for r in range(0, rows_pt, zr):
            pltpu.sync_copy(zbuf, accum.at[pl.ds(row0 + r, zr)])
        if tail:
            @pl.when(s == NS - 1)
            def _():
                pltpu.sync_copy(zbuf.at[pl.ds(0, tail)],
                                accum.at[pl.ds(NS * rows_pt, tail)])
        plsc.subcore_barrier()

        base0 = (c * NS + s) * epw
        def body(j, _):
            b = base0 + j * chunk
            pltpu.sync_copy(src_hbm.at[pl.ds(b, chunk)], sidx)
            pltpu.sync_copy(dst_hbm.at[pl.ds(b, chunk)], didx)
            pltpu.async_copy(table_hbm.at[sidx], rows, sem).wait()
            pltpu.sync_copy(rows, accum.at[didx], add=True)
            return 0
        lax.fori_loop(0, n_chunks, body, 0)
        plsc.subcore_barrier()

        pltpu.sync_copy(accum.at[pl.ds(row0, rows_pt)],
                        out_hbm.at[c, pl.ds(row0, rows_pt)])
        if tail:
            @pl.when(s == NS - 1)
            def _():
                pltpu.sync_copy(accum.at[pl.ds(NS * rows_pt, tail)],
                                out_hbm.at[c, pl.ds(NS * rows_pt, tail)])

    return k(src, dst, table)


def kernel(input_idx, edge_index, img_features, rel_features, att_features,
           name_features, char_features, entity_emb, gc1_w, gc1_b, gc2_w,
           gc2_b, img_w, img_b, rel_w, rel_b, att_w, att_b, name_w, name_b,
           char_w, char_b, fusion_weight):
    n = entity_emb.shape[0]
    x = jnp.take(entity_emb, input_idx, axis=0)
    src = edge_index[0]
    dst = edge_index[1]

    z1 = _matmul(x, gc1_w, gc1_b, bm=1000)
    p1 = _spmm_sc(src, dst, z1)
    z2 = _relu_partials_matmul(p1, gc2_w, gc2_b, bm=1000)
    p2 = _spmm_sc(src, dst, z2)
    gph = _scaled_partials(p2, fusion_weight, bm=1000)

    proj = _projections(
        img_features, rel_features, att_features, name_features,
        char_features, img_w, img_b, rel_w, rel_b, att_w, att_b,
        name_w, name_b, char_w, char_b, fusion_weight, bm=1000)

    return jnp.concatenate([gph, proj.reshape(n, -1)], axis=-1)
